# baseline (speedup 1.0000x reference)
import functools

import jax
import jax.numpy as jnp
from jax import lax
from jax.experimental import pallas as pl
from jax.experimental.pallas import tpu as pltpu
from jax.experimental.pallas import tpu_sc as plsc

MODEL_DIM = 64
SCALE = 8.0

NUM_CORES = 2
NUM_SUBCORES = 16
NUM_WORKERS = NUM_CORES * NUM_SUBCORES

BATCH = 4096
SEQ = 200
N_ROWS = BATCH * SEQ
LANES = 16
BG = BATCH // 128
N_UNITS = SEQ * BG
UNITS_PER_WORKER = N_UNITS // NUM_WORKERS
IDX_PER_WORKER = UNITS_PER_WORKER * 128


@functools.partial(
    pl.kernel,
    out_type=jax.ShapeDtypeStruct((SEQ, 8, BG, 8, 128), jnp.float32),
    mesh=plsc.VectorSubcoreMesh(core_axis_name="c", subcore_axis_name="s"),
    compiler_params=pltpu.CompilerParams(
        use_tc_tiling_on_sc=False, needs_layout_passes=False),
    scratch_types=[
        pltpu.VMEM((IDX_PER_WORKER,), jnp.int32),
        pltpu.VMEM((128, MODEL_DIM), jnp.float32),
        pltpu.VMEM((128, MODEL_DIM), jnp.float32),
        pltpu.VMEM((8, 8, 128), jnp.float32),
        pltpu.VMEM((8, 8, 128), jnp.float32),
        pltpu.SemaphoreType.DMA,
        pltpu.SemaphoreType.DMA,
        pltpu.SemaphoreType.DMA,
        pltpu.SemaphoreType.DMA,
    ],
)
def _emb_lookup(table_hbm, idx_hbm, out_hbm, idx_v, buf0, buf1, tb0, tb1,
                gsem0, gsem1, wsem0, wsem1):
    wid = lax.axis_index("s") * NUM_CORES + lax.axis_index("c")
    ubase = wid * UNITS_PER_WORKER
    pltpu.sync_copy(idx_hbm.at[pl.ds(ubase * 128, IDX_PER_WORKER)], idx_v)

    row_sel = [lax.iota(jnp.int32, LANES) + c * LANES for c in range(8)]

    def gather(i, buf, sem):
        pltpu.async_copy(table_hbm.at[idx_v.at[pl.ds(i * 128, 128)]],
                         buf, sem)

    def wait_gather(buf, sem):
        pltpu.make_async_copy(table_hbm.at[idx_v.at[pl.ds(0, 128)]],
                              buf, sem).wait()

    def transpose_scale(buf, tb):
        def step(d, _):
            dg = d // 8
            dr = lax.rem(d, 8)
            col = jnp.full((LANES,), d, dtype=jnp.int32)
            for c in range(8):
                v = plsc.load_gather(buf, [row_sel[c], col])
                tb[dg, dr, pl.ds(c * LANES, LANES)] = v * SCALE
            return 0
        lax.fori_loop(0, MODEL_DIM, step, 0, unroll=2)

    def writeback(i, tb, sem):
        u = ubase + i
        t = u // BG
        bg = lax.rem(u, BG)
        pltpu.async_copy(tb, out_hbm.at[t, :, bg], sem)

    def wait_writeback(tb, sem):
        pltpu.make_async_copy(tb, out_hbm.at[0, :, 0], sem).wait()

    gather(0, buf0, gsem0)
    gather(1, buf1, gsem1)

    def body(i, _):
        i0 = 2 * i
        i1 = i0 + 1
        wait_gather(buf0, gsem0)
        transpose_scale(buf0, tb0)
        writeback(i0, tb0, wsem0)
        wait_gather(buf1, gsem1)
        transpose_scale(buf1, tb1)
        writeback(i1, tb1, wsem1)

        @pl.when(i0 + 2 < UNITS_PER_WORKER)
        def _():
            gather(i0 + 2, buf0, gsem0)
            gather(i1 + 2, buf1, gsem1)
        wait_writeback(tb0, wsem0)
        wait_writeback(tb1, wsem1)
        return 0

    lax.fori_loop(0, UNITS_PER_WORKER // 2, body, 0)


def kernel(inputs, embeddings):
    idx = inputs.T.reshape(-1)
    out5 = _emb_lookup(embeddings, idx)
    return out5.transpose(2, 4, 0, 1, 3).reshape(BATCH, SEQ, MODEL_DIM)


# speedup vs baseline: 1.3997x; 1.3997x over previous
import functools

import jax
import jax.numpy as jnp
from jax import lax
from jax.experimental import pallas as pl
from jax.experimental.pallas import tpu as pltpu
from jax.experimental.pallas import tpu_sc as plsc

MODEL_DIM = 64
SCALE = 8.0

NUM_CORES = 2
NUM_SUBCORES = 16
NUM_WORKERS = NUM_CORES * NUM_SUBCORES

BATCH = 4096
SEQ = 200
N_ROWS = BATCH * SEQ
ROWS_PER_WORKER = N_ROWS // NUM_WORKERS
CHUNK = 512
N_CHUNKS = ROWS_PER_WORKER // CHUNK


@functools.partial(
    pl.kernel,
    out_type=jax.ShapeDtypeStruct((N_ROWS, MODEL_DIM), jnp.float32),
    mesh=plsc.VectorSubcoreMesh(core_axis_name="c", subcore_axis_name="s"),
    compiler_params=pltpu.CompilerParams(
        use_tc_tiling_on_sc=False, needs_layout_passes=False),
    scratch_types=[
        pltpu.VMEM((ROWS_PER_WORKER,), jnp.int32),
        pltpu.VMEM((CHUNK, MODEL_DIM), jnp.float32),
        pltpu.VMEM((CHUNK, MODEL_DIM), jnp.float32),
        pltpu.SemaphoreType.DMA,
        pltpu.SemaphoreType.DMA,
        pltpu.SemaphoreType.DMA,
        pltpu.SemaphoreType.DMA,
    ],
)
def _emb_lookup(table_hbm, idx_hbm, out_hbm, idx_v, buf0, buf1,
                gsem0, gsem1, wsem0, wsem1):
    wid = lax.axis_index("s") * NUM_CORES + lax.axis_index("c")
    base = wid * ROWS_PER_WORKER
    pltpu.sync_copy(idx_hbm.at[pl.ds(base, ROWS_PER_WORKER)], idx_v)

    def gather(c, buf, sem):
        pltpu.async_copy(table_hbm.at[idx_v.at[pl.ds(c * CHUNK, CHUNK)]],
                         buf, sem)

    def wait_gather(buf, sem):
        pltpu.make_async_copy(table_hbm.at[idx_v.at[pl.ds(0, CHUNK)]],
                              buf, sem).wait()

    def writeback(c, buf, sem):
        pltpu.async_copy(buf, out_hbm.at[pl.ds(base + c * CHUNK, CHUNK)], sem)

    def wait_writeback(buf, sem):
        pltpu.make_async_copy(buf, out_hbm.at[pl.ds(0, CHUNK)], sem).wait()

    gather(0, buf0, gsem0)
    gather(1, buf1, gsem1)

    def body(i, _):
        c0 = 2 * i
        c1 = c0 + 1
        wait_gather(buf0, gsem0)
        writeback(c0, buf0, wsem0)
        wait_gather(buf1, gsem1)
        writeback(c1, buf1, wsem1)

        @pl.when(c0 + 2 < N_CHUNKS)
        def _():
            wait_writeback(buf0, wsem0)
            gather(c0 + 2, buf0, gsem0)
            wait_writeback(buf1, wsem1)
            gather(c1 + 2, buf1, gsem1)
        return 0

    lax.fori_loop(0, N_CHUNKS // 2, body, 0)
    wait_writeback(buf0, wsem0)
    wait_writeback(buf1, wsem1)


def kernel(inputs, embeddings):
    idx = inputs.T.reshape(-1)
    out = _emb_lookup(embeddings, idx)
    out = out.reshape(SEQ, BATCH, MODEL_DIM) * SCALE
    return out.transpose(1, 0, 2)
